# baseline (speedup 1.0000x reference)
import functools

import jax
import jax.numpy as jnp
from jax import lax
from jax.experimental import pallas as pl
from jax.experimental.pallas import tpu as pltpu
from jax.experimental.pallas import tpu_sc as plsc

N_E = 8192
E_DIM = 32
N_TOK = 8192
BETA = 0.25

RA = 1024
CA = 1024


def _argmin_body(z_ref, emb_ref, idx_ref, bestv, besti):
    c = pl.program_id(1)
    nc = pl.num_programs(1)

    @pl.when(c == 0)
    def _():
        bestv[...] = jnp.full_like(bestv, jnp.inf)
        besti[...] = jnp.zeros_like(besti)

    zb = z_ref[...]
    eb = emb_ref[...]
    m = jax.lax.dot_general(zb.astype(jnp.bfloat16), eb, (((1,), (1,)), ((), ())),
                            preferred_element_type=jnp.float32)
    s1 = jnp.sum(zb * zb, axis=1, keepdims=True)
    s2 = jnp.sum(eb * eb, axis=1)
    d = (s1 + s2[None, :]) - 2.0 * m
    bmin = jnp.min(d, axis=1, keepdims=True)
    ids = jax.lax.broadcasted_iota(jnp.int32, (RA, CA), 1)
    big = jnp.int32(2**31 - 1)
    bidx = jnp.min(jnp.where(d == bmin, ids, big), axis=1, keepdims=True) + c * CA
    upd = bmin < bestv[...]
    besti[...] = jnp.where(upd, bidx, besti[...])
    bestv[...] = jnp.where(upd, bmin, bestv[...])

    @pl.when(c == nc - 1)
    def _():
        idx_ref[...] = besti[...]


def _argmin_call(z_flat, emb_w):
    grid = (N_TOK // RA, N_E // CA)
    return pl.pallas_call(
        _argmin_body,
        grid=grid,
        in_specs=[
            pl.BlockSpec((RA, E_DIM), lambda r, c: (r, 0)),
            pl.BlockSpec((CA, E_DIM), lambda r, c: (c, 0)),
        ],
        out_specs=pl.BlockSpec((RA, 1), lambda r, c: (r, 0)),
        out_shape=jax.ShapeDtypeStruct((N_TOK, 1), jnp.int32),
        scratch_shapes=[
            pltpu.VMEM((RA, 1), jnp.float32),
            pltpu.VMEM((RA, 1), jnp.int32),
        ],
    )(z_flat, emb_w)


_SC_GATHER = None


def _sc_gather():
    global _SC_GATHER
    if _SC_GATHER is not None:
        return _SC_GATHER
    info = plsc.get_sparse_core_info()
    nc, ns = info.num_cores, info.num_subcores
    nw = nc * ns
    b_per_w = N_TOK // nw
    mesh = plsc.VectorSubcoreMesh(core_axis_name="c", subcore_axis_name="s")

    @functools.partial(
        pl.kernel, mesh=mesh,
        out_type=jax.ShapeDtypeStruct((N_TOK, 128), jnp.float32),
        scratch_types=[
            pltpu.VMEM((b_per_w,), jnp.int32),
            pltpu.VMEM((b_per_w, 128), jnp.float32),
            pltpu.SemaphoreType.DMA,
        ],
    )
    def gather_k(table_hbm, idx_hbm, out_hbm, idx_v, rows_v, sem):
        wid = lax.axis_index("s") * nc + lax.axis_index("c")
        base = wid * b_per_w
        pltpu.sync_copy(idx_hbm.at[pl.ds(base, b_per_w)], idx_v)
        pltpu.async_copy(table_hbm.at[idx_v], rows_v, sem).wait()
        pltpu.sync_copy(rows_v, out_hbm.at[pl.ds(base, b_per_w)])

    _SC_GATHER = gather_k
    return gather_k


RB = 256


def _onehot_body(idx_ref, z_ref, zq_ref,
                 enc_ref, zqo_ref, hist_ref, loss_ref, perp_ref):
    r = pl.program_id(0)
    nr = pl.num_programs(0)

    @pl.when(r == 0)
    def _():
        hist_ref[...] = jnp.zeros_like(hist_ref)
        loss_ref[0, 0] = 0.0

    idx = idx_ref[...]
    ids = jax.lax.broadcasted_iota(jnp.int32, (RB, N_E), 1)
    oh = jnp.where(ids == idx, 1.0, 0.0).astype(jnp.float32)
    enc_ref[...] = oh
    zq = zq_ref[...]
    hist_ref[...] += jnp.sum(oh, axis=0).reshape(N_E // 1024, 1024)
    zb = z_ref[...]
    zqo_ref[...] = zb + (zq - zb)
    loss_ref[0, 0] += jnp.sum((zq - zb) ** 2)

    @pl.when(r == nr - 1)
    def _():
        loss_ref[0, 0] = loss_ref[0, 0] * ((1.0 + BETA) / (N_TOK * E_DIM))
        em = hist_ref[...] * (1.0 / N_TOK)
        perp_ref[0, 0] = jnp.exp(-jnp.sum(em * jnp.log(em + 1e-10)))


def _onehot_call(idx, z_flat, zq_flat):
    grid = (N_TOK // RB,)
    return pl.pallas_call(
        _onehot_body,
        grid=grid,
        in_specs=[
            pl.BlockSpec((RB, 1), lambda r: (r, 0)),
            pl.BlockSpec((RB, E_DIM), lambda r: (r, 0)),
            pl.BlockSpec((RB, E_DIM), lambda r: (r, 0)),
        ],
        out_specs=[
            pl.BlockSpec((RB, N_E), lambda r: (r, 0)),
            pl.BlockSpec((RB, E_DIM), lambda r: (r, 0)),
            pl.BlockSpec((N_E // 1024, 1024), lambda r: (0, 0)),
            pl.BlockSpec(memory_space=pltpu.SMEM),
            pl.BlockSpec(memory_space=pltpu.SMEM),
        ],
        out_shape=[
            jax.ShapeDtypeStruct((N_TOK, N_E), jnp.float32),
            jax.ShapeDtypeStruct((N_TOK, E_DIM), jnp.float32),
            jax.ShapeDtypeStruct((N_E // 1024, 1024), jnp.float32),
            jax.ShapeDtypeStruct((1, 1), jnp.float32),
            jax.ShapeDtypeStruct((1, 1), jnp.float32),
        ],
    )(idx, z_flat, zq_flat)


def kernel(z, emb_w):
    zp = jnp.transpose(z, (0, 2, 3, 1))
    z_flat = zp.reshape(-1, E_DIM)
    idx = _argmin_call(z_flat, emb_w)
    emb_pad = jnp.pad(emb_w, ((0, 0), (0, 128 - E_DIM)))
    zq_flat = _sc_gather()(emb_pad, idx.reshape(-1))[:, :E_DIM]
    enc, zq_out, _hist, loss, perp = _onehot_call(idx, z_flat, zq_flat)
    z_q = zq_out.reshape(zp.shape).transpose(0, 3, 1, 2)
    return (loss[0, 0], z_q, perp[0, 0], enc, idx)


# speedup vs baseline: 1.0017x; 1.0017x over previous
import functools

import jax
import jax.numpy as jnp
from jax import lax
from jax.experimental import pallas as pl
from jax.experimental.pallas import tpu as pltpu
from jax.experimental.pallas import tpu_sc as plsc

N_E = 8192
E_DIM = 32
N_TOK = 8192
BETA = 0.25

RA = 1024
CA = 1024


def _argmin_body(z_ref, emb_ref, idx_ref, bestv, besti):
    c = pl.program_id(1)
    nc = pl.num_programs(1)

    @pl.when(c == 0)
    def _():
        bestv[...] = jnp.full_like(bestv, jnp.inf)
        besti[...] = jnp.zeros_like(besti)

    zb = z_ref[...]
    eb = emb_ref[...]
    m = jax.lax.dot_general(zb.astype(jnp.bfloat16), eb, (((1,), (1,)), ((), ())),
                            preferred_element_type=jnp.float32)
    s1 = jnp.sum(zb * zb, axis=1, keepdims=True)
    s2 = jnp.sum(eb * eb, axis=1)
    d = (s1 + s2[None, :]) - 2.0 * m
    bmin = jnp.min(d, axis=1, keepdims=True)
    ids = jax.lax.broadcasted_iota(jnp.int32, (RA, CA), 1)
    big = jnp.int32(2**31 - 1)
    bidx = jnp.min(jnp.where(d == bmin, ids, big), axis=1, keepdims=True) + c * CA
    upd = bmin < bestv[...]
    besti[...] = jnp.where(upd, bidx, besti[...])
    bestv[...] = jnp.where(upd, bmin, bestv[...])

    @pl.when(c == nc - 1)
    def _():
        idx_ref[...] = besti[...]


def _argmin_call(z_flat, emb_w):
    grid = (N_TOK // RA, N_E // CA)
    return pl.pallas_call(
        _argmin_body,
        grid=grid,
        in_specs=[
            pl.BlockSpec((RA, E_DIM), lambda r, c: (r, 0)),
            pl.BlockSpec((CA, E_DIM), lambda r, c: (c, 0)),
        ],
        out_specs=pl.BlockSpec((RA, 1), lambda r, c: (r, 0)),
        out_shape=jax.ShapeDtypeStruct((N_TOK, 1), jnp.int32),
        scratch_shapes=[
            pltpu.VMEM((RA, 1), jnp.float32),
            pltpu.VMEM((RA, 1), jnp.int32),
        ],
    )(z_flat, emb_w)


_SC_GATHER = None


def _sc_gather():
    global _SC_GATHER
    if _SC_GATHER is not None:
        return _SC_GATHER
    info = plsc.get_sparse_core_info()
    nc, ns = info.num_cores, info.num_subcores
    nw = nc * ns
    b_per_w = N_TOK // nw
    mesh = plsc.VectorSubcoreMesh(core_axis_name="c", subcore_axis_name="s")

    @functools.partial(
        pl.kernel, mesh=mesh,
        out_type=jax.ShapeDtypeStruct((N_TOK, 128), jnp.float32),
        scratch_types=[
            pltpu.VMEM((b_per_w,), jnp.int32),
            pltpu.VMEM((b_per_w, 128), jnp.float32),
            pltpu.SemaphoreType.DMA,
        ],
    )
    def gather_k(table_hbm, idx_hbm, out_hbm, idx_v, rows_v, sem):
        wid = lax.axis_index("s") * nc + lax.axis_index("c")
        base = wid * b_per_w
        pltpu.sync_copy(idx_hbm.at[pl.ds(base, b_per_w)], idx_v)
        pltpu.async_copy(table_hbm.at[idx_v], rows_v, sem).wait()
        pltpu.sync_copy(rows_v, out_hbm.at[pl.ds(base, b_per_w)])

    _SC_GATHER = gather_k
    return gather_k


RB = 256


def _onehot_body(idx_ref, enc_ref, hist_ref):
    r = pl.program_id(0)

    @pl.when(r == 0)
    def _():
        hist_ref[...] = jnp.zeros_like(hist_ref)

    idx = idx_ref[...]
    ids = jax.lax.broadcasted_iota(jnp.int32, (RB, N_E), 1)
    oh = jnp.where(ids == idx, 1.0, 0.0).astype(jnp.float32)
    enc_ref[...] = oh
    hist_ref[...] += jnp.sum(oh, axis=0).reshape(N_E // 1024, 1024)


def _onehot_call(idx):
    grid = (N_TOK // RB,)
    return pl.pallas_call(
        _onehot_body,
        grid=grid,
        in_specs=[pl.BlockSpec((RB, 1), lambda r: (r, 0))],
        out_specs=[
            pl.BlockSpec((RB, N_E), lambda r: (r, 0)),
            pl.BlockSpec((N_E // 1024, 1024), lambda r: (0, 0)),
        ],
        out_shape=[
            jax.ShapeDtypeStruct((N_TOK, N_E), jnp.float32),
            jax.ShapeDtypeStruct((N_E // 1024, 1024), jnp.float32),
        ],
    )(idx)


def _final_body(zq_ref, z_ref, hist_ref, zqo_ref, loss_ref, perp_ref):
    zq = zq_ref[...]
    zb = z_ref[...]
    zqo_ref[...] = zb + (zq - zb)
    loss_ref[0, 0] = jnp.sum((zq - zb) ** 2) * ((1.0 + BETA) / (N_TOK * E_DIM))
    em = hist_ref[...] * (1.0 / N_TOK)
    perp_ref[0, 0] = jnp.exp(-jnp.sum(em * jnp.log(em + 1e-10)))


def _final_call(zq_flat, z_flat, hist):
    return pl.pallas_call(
        _final_body,
        out_specs=[
            pl.BlockSpec((N_TOK, E_DIM), lambda: (0, 0)),
            pl.BlockSpec(memory_space=pltpu.SMEM),
            pl.BlockSpec(memory_space=pltpu.SMEM),
        ],
        out_shape=[
            jax.ShapeDtypeStruct((N_TOK, E_DIM), jnp.float32),
            jax.ShapeDtypeStruct((1, 1), jnp.float32),
            jax.ShapeDtypeStruct((1, 1), jnp.float32),
        ],
    )(zq_flat, z_flat, hist)


def kernel(z, emb_w):
    zp = jnp.transpose(z, (0, 2, 3, 1))
    z_flat = zp.reshape(-1, E_DIM)
    idx = _argmin_call(z_flat, emb_w)
    emb_pad = jnp.pad(emb_w, ((0, 0), (0, 128 - E_DIM)))
    zq_flat = _sc_gather()(emb_pad, idx.reshape(-1))[:, :E_DIM]
    enc, hist = _onehot_call(idx)
    zq_out, loss, perp = _final_call(zq_flat, z_flat, hist)
    z_q = zq_out.reshape(zp.shape).transpose(0, 3, 1, 2)
    return (loss[0, 0], z_q, perp[0, 0], enc, idx)


# speedup vs baseline: 1.0852x; 1.0833x over previous
import functools

import jax
import jax.numpy as jnp
from jax import lax
from jax.experimental import pallas as pl
from jax.experimental.pallas import tpu as pltpu
from jax.experimental.pallas import tpu_sc as plsc

N_E = 8192
E_DIM = 32
N_TOK = 8192
BETA = 0.25

RA = 1024
CA = 2048


def _argmin_body(z_ref, emb_ref, idx_ref, bestv, besti):
    c = pl.program_id(1)
    nc = pl.num_programs(1)

    @pl.when(c == 0)
    def _():
        bestv[...] = jnp.full_like(bestv, jnp.inf)
        besti[...] = jnp.zeros_like(besti)

    zb = z_ref[...]
    eb = emb_ref[...]
    m = jax.lax.dot_general(zb.astype(jnp.bfloat16), eb, (((1,), (1,)), ((), ())),
                            preferred_element_type=jnp.float32)
    s1 = jnp.sum(zb * zb, axis=1, keepdims=True)
    s2 = jnp.sum(eb * eb, axis=1)
    d = (s1 + s2[None, :]) - 2.0 * m
    bmin = jnp.min(d, axis=1, keepdims=True)
    ids = jax.lax.broadcasted_iota(jnp.int32, (RA, CA), 1)
    big = jnp.int32(2**31 - 1)
    bidx = jnp.min(jnp.where(d == bmin, ids, big), axis=1, keepdims=True) + c * CA
    upd = bmin < bestv[...]
    besti[...] = jnp.where(upd, bidx, besti[...])
    bestv[...] = jnp.where(upd, bmin, bestv[...])

    @pl.when(c == nc - 1)
    def _():
        idx_ref[...] = besti[...]


def _argmin_call(z_flat, emb_w):
    grid = (N_TOK // RA, N_E // CA)
    return pl.pallas_call(
        _argmin_body,
        grid=grid,
        in_specs=[
            pl.BlockSpec((RA, E_DIM), lambda r, c: (r, 0)),
            pl.BlockSpec((CA, E_DIM), lambda r, c: (c, 0)),
        ],
        out_specs=pl.BlockSpec((RA, 1), lambda r, c: (r, 0)),
        out_shape=jax.ShapeDtypeStruct((N_TOK, 1), jnp.int32),
        scratch_shapes=[
            pltpu.VMEM((RA, 1), jnp.float32),
            pltpu.VMEM((RA, 1), jnp.int32),
        ],
    )(z_flat, emb_w)


_SC_GATHER = None


def _sc_gather():
    global _SC_GATHER
    if _SC_GATHER is not None:
        return _SC_GATHER
    info = plsc.get_sparse_core_info()
    nc, ns = info.num_cores, info.num_subcores
    nw = nc * ns
    b_per_w = N_TOK // nw
    mesh = plsc.VectorSubcoreMesh(core_axis_name="c", subcore_axis_name="s")

    @functools.partial(
        pl.kernel, mesh=mesh,
        out_type=jax.ShapeDtypeStruct((N_TOK, 128), jnp.float32),
        scratch_types=[
            pltpu.VMEM((b_per_w,), jnp.int32),
            pltpu.VMEM((b_per_w, 128), jnp.float32),
            pltpu.SemaphoreType.DMA,
        ],
    )
    def gather_k(table_hbm, idx_hbm, out_hbm, idx_v, rows_v, sem):
        wid = lax.axis_index("s") * nc + lax.axis_index("c")
        base = wid * b_per_w
        pltpu.sync_copy(idx_hbm.at[pl.ds(base, b_per_w)], idx_v)
        pltpu.async_copy(table_hbm.at[idx_v], rows_v, sem).wait()
        pltpu.sync_copy(rows_v, out_hbm.at[pl.ds(base, b_per_w)])

    _SC_GATHER = gather_k
    return gather_k


RB = 256


def _onehot_body(idx_ref, enc_ref, hist_ref):
    r = pl.program_id(0)

    @pl.when(r == 0)
    def _():
        hist_ref[...] = jnp.zeros_like(hist_ref)

    idx = idx_ref[...]
    ids = jax.lax.broadcasted_iota(jnp.int32, (RB, N_E), 1)
    oh = jnp.where(ids == idx, 1.0, 0.0).astype(jnp.float32)
    enc_ref[...] = oh
    hist_ref[...] += jnp.sum(oh, axis=0).reshape(N_E // 1024, 1024)


def _onehot_call(idx):
    grid = (N_TOK // RB,)
    return pl.pallas_call(
        _onehot_body,
        grid=grid,
        in_specs=[pl.BlockSpec((RB, 1), lambda r: (r, 0))],
        out_specs=[
            pl.BlockSpec((RB, N_E), lambda r: (r, 0)),
            pl.BlockSpec((N_E // 1024, 1024), lambda r: (0, 0)),
        ],
        out_shape=[
            jax.ShapeDtypeStruct((N_TOK, N_E), jnp.float32),
            jax.ShapeDtypeStruct((N_E // 1024, 1024), jnp.float32),
        ],
    )(idx)


def _final_body(zq_ref, z_ref, hist_ref, zqo_ref, loss_ref, perp_ref):
    zq = zq_ref[...]
    zb = z_ref[...]
    zqo_ref[...] = zb + (zq - zb)
    loss_ref[0, 0] = jnp.sum((zq - zb) ** 2) * ((1.0 + BETA) / (N_TOK * E_DIM))
    em = hist_ref[...] * (1.0 / N_TOK)
    perp_ref[0, 0] = jnp.exp(-jnp.sum(em * jnp.log(em + 1e-10)))


def _final_call(zq_flat, z_flat, hist):
    return pl.pallas_call(
        _final_body,
        out_specs=[
            pl.BlockSpec((N_TOK, E_DIM), lambda: (0, 0)),
            pl.BlockSpec(memory_space=pltpu.SMEM),
            pl.BlockSpec(memory_space=pltpu.SMEM),
        ],
        out_shape=[
            jax.ShapeDtypeStruct((N_TOK, E_DIM), jnp.float32),
            jax.ShapeDtypeStruct((1, 1), jnp.float32),
            jax.ShapeDtypeStruct((1, 1), jnp.float32),
        ],
    )(zq_flat, z_flat, hist)


def kernel(z, emb_w):
    zp = jnp.transpose(z, (0, 2, 3, 1))
    z_flat = zp.reshape(-1, E_DIM)
    idx = _argmin_call(z_flat, emb_w)
    emb_pad = jnp.pad(emb_w, ((0, 0), (0, 128 - E_DIM)))
    zq_flat = _sc_gather()(emb_pad, idx.reshape(-1))[:, :E_DIM]
    enc, hist = _onehot_call(idx)
    zq_out, loss, perp = _final_call(zq_flat, z_flat, hist)
    z_q = zq_out.reshape(zp.shape).transpose(0, 3, 1, 2)
    return (loss[0, 0], z_q, perp[0, 0], enc, idx)


# speedup vs baseline: 1.0964x; 1.0104x over previous
import functools

import jax
import jax.numpy as jnp
from jax import lax
from jax.experimental import pallas as pl
from jax.experimental.pallas import tpu as pltpu
from jax.experimental.pallas import tpu_sc as plsc

N_E = 8192
E_DIM = 32
N_TOK = 8192
BETA = 0.25

RA = 512
CA = 4096


def _argmin_body(z_ref, emb_ref, idx_ref, bestv, besti):
    c = pl.program_id(1)
    nc = pl.num_programs(1)

    @pl.when(c == 0)
    def _():
        bestv[...] = jnp.full_like(bestv, jnp.inf)
        besti[...] = jnp.zeros_like(besti)

    zb = z_ref[...]
    eb = emb_ref[...]
    m = jax.lax.dot_general(zb.astype(jnp.bfloat16), eb, (((1,), (1,)), ((), ())),
                            preferred_element_type=jnp.float32)
    s1 = jnp.sum(zb * zb, axis=1, keepdims=True)
    s2 = jnp.sum(eb * eb, axis=1)
    d = (s1 + s2[None, :]) - 2.0 * m
    bmin = jnp.min(d, axis=1, keepdims=True)
    ids = jax.lax.broadcasted_iota(jnp.int32, (RA, CA), 1)
    big = jnp.int32(2**31 - 1)
    bidx = jnp.min(jnp.where(d == bmin, ids, big), axis=1, keepdims=True) + c * CA
    upd = bmin < bestv[...]
    besti[...] = jnp.where(upd, bidx, besti[...])
    bestv[...] = jnp.where(upd, bmin, bestv[...])

    @pl.when(c == nc - 1)
    def _():
        idx_ref[...] = besti[...]


def _argmin_call(z_flat, emb_w):
    grid = (N_TOK // RA, N_E // CA)
    return pl.pallas_call(
        _argmin_body,
        grid=grid,
        in_specs=[
            pl.BlockSpec((RA, E_DIM), lambda r, c: (r, 0)),
            pl.BlockSpec((CA, E_DIM), lambda r, c: (c, 0)),
        ],
        out_specs=pl.BlockSpec((RA, 1), lambda r, c: (r, 0)),
        out_shape=jax.ShapeDtypeStruct((N_TOK, 1), jnp.int32),
        scratch_shapes=[
            pltpu.VMEM((RA, 1), jnp.float32),
            pltpu.VMEM((RA, 1), jnp.int32),
        ],
    )(z_flat, emb_w)


_SC_GATHER = None


def _sc_gather():
    global _SC_GATHER
    if _SC_GATHER is not None:
        return _SC_GATHER
    info = plsc.get_sparse_core_info()
    nc, ns = info.num_cores, info.num_subcores
    nw = nc * ns
    b_per_w = N_TOK // nw
    mesh = plsc.VectorSubcoreMesh(core_axis_name="c", subcore_axis_name="s")

    @functools.partial(
        pl.kernel, mesh=mesh,
        out_type=jax.ShapeDtypeStruct((N_TOK, 128), jnp.float32),
        scratch_types=[
            pltpu.VMEM((b_per_w,), jnp.int32),
            pltpu.VMEM((b_per_w, 128), jnp.float32),
            pltpu.SemaphoreType.DMA,
        ],
    )
    def gather_k(table_hbm, idx_hbm, out_hbm, idx_v, rows_v, sem):
        wid = lax.axis_index("s") * nc + lax.axis_index("c")
        base = wid * b_per_w
        pltpu.sync_copy(idx_hbm.at[pl.ds(base, b_per_w)], idx_v)
        pltpu.async_copy(table_hbm.at[idx_v], rows_v, sem).wait()
        pltpu.sync_copy(rows_v, out_hbm.at[pl.ds(base, b_per_w)])

    _SC_GATHER = gather_k
    return gather_k


RB = 256


def _onehot_body(idx_ref, enc_ref, hist_ref):
    r = pl.program_id(0)

    @pl.when(r == 0)
    def _():
        hist_ref[...] = jnp.zeros_like(hist_ref)

    idx = idx_ref[...]
    ids = jax.lax.broadcasted_iota(jnp.int32, (RB, N_E), 1)
    oh = jnp.where(ids == idx, 1.0, 0.0).astype(jnp.float32)
    enc_ref[...] = oh
    hist_ref[...] += jnp.sum(oh, axis=0).reshape(N_E // 1024, 1024)


def _onehot_call(idx):
    grid = (N_TOK // RB,)
    return pl.pallas_call(
        _onehot_body,
        grid=grid,
        in_specs=[pl.BlockSpec((RB, 1), lambda r: (r, 0))],
        out_specs=[
            pl.BlockSpec((RB, N_E), lambda r: (r, 0)),
            pl.BlockSpec((N_E // 1024, 1024), lambda r: (0, 0)),
        ],
        out_shape=[
            jax.ShapeDtypeStruct((N_TOK, N_E), jnp.float32),
            jax.ShapeDtypeStruct((N_E // 1024, 1024), jnp.float32),
        ],
    )(idx)


def _final_body(zq_ref, z_ref, hist_ref, zqo_ref, loss_ref, perp_ref):
    zq = zq_ref[...]
    zb = z_ref[...]
    zqo_ref[...] = zb + (zq - zb)
    loss_ref[0, 0] = jnp.sum((zq - zb) ** 2) * ((1.0 + BETA) / (N_TOK * E_DIM))
    em = hist_ref[...] * (1.0 / N_TOK)
    perp_ref[0, 0] = jnp.exp(-jnp.sum(em * jnp.log(em + 1e-10)))


def _final_call(zq_flat, z_flat, hist):
    return pl.pallas_call(
        _final_body,
        out_specs=[
            pl.BlockSpec((N_TOK, E_DIM), lambda: (0, 0)),
            pl.BlockSpec(memory_space=pltpu.SMEM),
            pl.BlockSpec(memory_space=pltpu.SMEM),
        ],
        out_shape=[
            jax.ShapeDtypeStruct((N_TOK, E_DIM), jnp.float32),
            jax.ShapeDtypeStruct((1, 1), jnp.float32),
            jax.ShapeDtypeStruct((1, 1), jnp.float32),
        ],
    )(zq_flat, z_flat, hist)


def kernel(z, emb_w):
    zp = jnp.transpose(z, (0, 2, 3, 1))
    z_flat = zp.reshape(-1, E_DIM)
    idx = _argmin_call(z_flat, emb_w)
    emb_pad = jnp.pad(emb_w, ((0, 0), (0, 128 - E_DIM)))
    zq_flat = _sc_gather()(emb_pad, idx.reshape(-1))[:, :E_DIM]
    enc, hist = _onehot_call(idx)
    zq_out, loss, perp = _final_call(zq_flat, z_flat, hist)
    z_q = zq_out.reshape(zp.shape).transpose(0, 3, 1, 2)
    return (loss[0, 0], z_q, perp[0, 0], enc, idx)


# speedup vs baseline: 1.1191x; 1.0207x over previous
import functools

import jax
import jax.numpy as jnp
from jax import lax
from jax.experimental import pallas as pl
from jax.experimental.pallas import tpu as pltpu
from jax.experimental.pallas import tpu_sc as plsc

N_E = 8192
E_DIM = 32
N_TOK = 8192
BETA = 0.25

RA = 512
CA = 8192


def _argmin_body(z_ref, emb_ref, idx_ref, bestv, besti):
    c = pl.program_id(1)
    nc = pl.num_programs(1)

    @pl.when(c == 0)
    def _():
        bestv[...] = jnp.full_like(bestv, jnp.inf)
        besti[...] = jnp.zeros_like(besti)

    zb = z_ref[...]
    eb = emb_ref[...]
    m = jax.lax.dot_general(zb.astype(jnp.bfloat16), eb, (((1,), (1,)), ((), ())),
                            preferred_element_type=jnp.float32)
    s1 = jnp.sum(zb * zb, axis=1, keepdims=True)
    s2 = jnp.sum(eb * eb, axis=1)
    d = (s1 + s2[None, :]) - 2.0 * m
    bmin = jnp.min(d, axis=1, keepdims=True)
    ids = jax.lax.broadcasted_iota(jnp.int32, (RA, CA), 1)
    big = jnp.int32(2**31 - 1)
    bidx = jnp.min(jnp.where(d == bmin, ids, big), axis=1, keepdims=True) + c * CA
    upd = bmin < bestv[...]
    besti[...] = jnp.where(upd, bidx, besti[...])
    bestv[...] = jnp.where(upd, bmin, bestv[...])

    @pl.when(c == nc - 1)
    def _():
        idx_ref[...] = besti[...]


def _argmin_call(z_flat, emb_w):
    grid = (N_TOK // RA, N_E // CA)
    return pl.pallas_call(
        _argmin_body,
        grid=grid,
        in_specs=[
            pl.BlockSpec((RA, E_DIM), lambda r, c: (r, 0)),
            pl.BlockSpec((CA, E_DIM), lambda r, c: (c, 0)),
        ],
        out_specs=pl.BlockSpec((RA, 1), lambda r, c: (r, 0)),
        out_shape=jax.ShapeDtypeStruct((N_TOK, 1), jnp.int32),
        scratch_shapes=[
            pltpu.VMEM((RA, 1), jnp.float32),
            pltpu.VMEM((RA, 1), jnp.int32),
        ],
    )(z_flat, emb_w)


_SC_GATHER = None


def _sc_gather():
    global _SC_GATHER
    if _SC_GATHER is not None:
        return _SC_GATHER
    info = plsc.get_sparse_core_info()
    nc, ns = info.num_cores, info.num_subcores
    nw = nc * ns
    b_per_w = N_TOK // nw
    mesh = plsc.VectorSubcoreMesh(core_axis_name="c", subcore_axis_name="s")

    @functools.partial(
        pl.kernel, mesh=mesh,
        out_type=jax.ShapeDtypeStruct((N_TOK, 128), jnp.float32),
        scratch_types=[
            pltpu.VMEM((b_per_w,), jnp.int32),
            pltpu.VMEM((b_per_w, 128), jnp.float32),
            pltpu.SemaphoreType.DMA,
        ],
    )
    def gather_k(table_hbm, idx_hbm, out_hbm, idx_v, rows_v, sem):
        wid = lax.axis_index("s") * nc + lax.axis_index("c")
        base = wid * b_per_w
        pltpu.sync_copy(idx_hbm.at[pl.ds(base, b_per_w)], idx_v)
        pltpu.async_copy(table_hbm.at[idx_v], rows_v, sem).wait()
        pltpu.sync_copy(rows_v, out_hbm.at[pl.ds(base, b_per_w)])

    _SC_GATHER = gather_k
    return gather_k


RB = 256


def _onehot_body(idx_ref, enc_ref, hist_ref):
    r = pl.program_id(0)

    @pl.when(r == 0)
    def _():
        hist_ref[...] = jnp.zeros_like(hist_ref)

    idx = idx_ref[...]
    ids = jax.lax.broadcasted_iota(jnp.int32, (RB, N_E), 1)
    oh = jnp.where(ids == idx, 1.0, 0.0).astype(jnp.float32)
    enc_ref[...] = oh
    hist_ref[...] += jnp.sum(oh, axis=0).reshape(N_E // 1024, 1024)


def _onehot_call(idx):
    grid = (N_TOK // RB,)
    return pl.pallas_call(
        _onehot_body,
        grid=grid,
        in_specs=[pl.BlockSpec((RB, 1), lambda r: (r, 0))],
        out_specs=[
            pl.BlockSpec((RB, N_E), lambda r: (r, 0)),
            pl.BlockSpec((N_E // 1024, 1024), lambda r: (0, 0)),
        ],
        out_shape=[
            jax.ShapeDtypeStruct((N_TOK, N_E), jnp.float32),
            jax.ShapeDtypeStruct((N_E // 1024, 1024), jnp.float32),
        ],
    )(idx)


def _final_body(zq_ref, z_ref, hist_ref, zqo_ref, loss_ref, perp_ref):
    zq = zq_ref[...]
    zb = z_ref[...]
    zqo_ref[...] = zb + (zq - zb)
    loss_ref[0, 0] = jnp.sum((zq - zb) ** 2) * ((1.0 + BETA) / (N_TOK * E_DIM))
    em = hist_ref[...] * (1.0 / N_TOK)
    perp_ref[0, 0] = jnp.exp(-jnp.sum(em * jnp.log(em + 1e-10)))


def _final_call(zq_flat, z_flat, hist):
    return pl.pallas_call(
        _final_body,
        out_specs=[
            pl.BlockSpec((N_TOK, E_DIM), lambda: (0, 0)),
            pl.BlockSpec(memory_space=pltpu.SMEM),
            pl.BlockSpec(memory_space=pltpu.SMEM),
        ],
        out_shape=[
            jax.ShapeDtypeStruct((N_TOK, E_DIM), jnp.float32),
            jax.ShapeDtypeStruct((1, 1), jnp.float32),
            jax.ShapeDtypeStruct((1, 1), jnp.float32),
        ],
    )(zq_flat, z_flat, hist)


def kernel(z, emb_w):
    zp = jnp.transpose(z, (0, 2, 3, 1))
    z_flat = zp.reshape(-1, E_DIM)
    idx = _argmin_call(z_flat, emb_w)
    emb_pad = jnp.pad(emb_w, ((0, 0), (0, 128 - E_DIM)))
    zq_flat = _sc_gather()(emb_pad, idx.reshape(-1))[:, :E_DIM]
    enc, hist = _onehot_call(idx)
    zq_out, loss, perp = _final_call(zq_flat, z_flat, hist)
    z_q = zq_out.reshape(zp.shape).transpose(0, 3, 1, 2)
    return (loss[0, 0], z_q, perp[0, 0], enc, idx)
